# Initial kernel scaffold; baseline (speedup 1.0000x reference)
import jax
import jax.numpy as jnp
from jax.experimental import pallas as pl


def kernel(f_atoms, f_bonds, b2a, b2revb, Wi_w, Wi_b, Wh_w, Wh_b, Wo_w, Wo_b):
    raise NotImplementedError("write your pallas kernel here")


# speedup vs baseline: 1.5310x; 1.5310x over previous
import functools

import jax
import jax.numpy as jnp
from jax import lax
from jax.experimental import pallas as pl
from jax.experimental.pallas import tpu as pltpu
from jax.experimental.pallas import tpu_sc as plsc

NB = 320000
NA = 10000
HID = 128
BFD = 16
AFD = 128

NC = 2
NS = 16
NW = NC * NS

CH = 64
NCHUNK = 160
BPT = NCHUNK * CH
NBP = NW * BPT
NPAD = NBP - NB
NA2 = 10016
DUMMY = NA

ROWS_PT = 624


@functools.cache
def _sc_mesh():
    return plsc.VectorSubcoreMesh(
        core_axis_name="c", subcore_axis_name="s",
        num_cores=NC, num_subcores=NS)


def _init_body(fb_ref, w_ref, b_ref, out_ref):
    acc = jnp.dot(fb_ref[...], w_ref[...], preferred_element_type=jnp.float32)
    out_ref[...] = jnp.maximum(acc + b_ref[...], 0.0)


def _tc_init(f_bonds, wiT, bi):
    bt = 4000
    grid = NB // bt
    return pl.pallas_call(
        _init_body,
        grid=(grid,),
        in_specs=[
            pl.BlockSpec((bt, BFD), lambda i: (i, 0)),
            pl.BlockSpec((BFD, HID), lambda i: (0, 0)),
            pl.BlockSpec((1, HID), lambda i: (0, 0)),
        ],
        out_specs=pl.BlockSpec((bt, HID), lambda i: (i, 0)),
        out_shape=jax.ShapeDtypeStruct((NBP, HID), jnp.float32),
    )(f_bonds, wiT, bi)


def _g_body(h_ref, w_ref, out_ref):
    out_ref[...] = jnp.dot(h_ref[...], w_ref[...],
                           preferred_element_type=jnp.float32)


def _tc_g(h_bonds, whT):
    bt = 4000
    grid = NB // bt
    return pl.pallas_call(
        _g_body,
        grid=(grid,),
        in_specs=[
            pl.BlockSpec((bt, HID), lambda i: (i, 0)),
            pl.BlockSpec((HID, HID), lambda i: (0, 0)),
        ],
        out_specs=pl.BlockSpec((bt, HID), lambda i: (i, 0)),
        out_shape=jax.ShapeDtypeStruct((NBP, HID), jnp.float32),
    )(h_bonds, whT)


def _a_body(sp_ref, w_ref, b_ref, out_ref):
    s = sp_ref[0] + sp_ref[1]
    out_ref[...] = jnp.dot(s, w_ref[...],
                           preferred_element_type=jnp.float32) + b_ref[...]


def _tc_a(s_part, whT, bh):
    return pl.pallas_call(
        _a_body,
        out_shape=jax.ShapeDtypeStruct((NA2, HID), jnp.float32),
    )(s_part, whT, bh)


def _final_body(fa_ref, sp_ref, w1_ref, w2_ref, b_ref, out_ref):
    s = sp_ref[0, :NA] + sp_ref[1, :NA]
    acc = jnp.dot(fa_ref[...], w1_ref[...], preferred_element_type=jnp.float32)
    acc += jnp.dot(s, w2_ref[...], preferred_element_type=jnp.float32)
    out_ref[...] = jnp.maximum(acc + b_ref[...], 0.0)


def _tc_final(f_atoms, s_part, wo1T, wo2T, bo):
    return pl.pallas_call(
        _final_body,
        out_shape=jax.ShapeDtypeStruct((NA, HID), jnp.float32),
    )(f_atoms, s_part, wo1T, wo2T, bo)


def _fill_zero_rows(buf, nrows):
    def row(r, c):
        for j in range(HID // 16):
            buf[r, pl.ds(j * 16, 16)] = jnp.zeros((16,), jnp.float32)
        return c
    lax.fori_loop(0, nrows, row, None)


def _zero_accum(s_sh, stage, sid):
    _fill_zero_rows(stage, CH)
    for j in range(ROWS_PT // CH):
        pltpu.sync_copy(stage, s_sh.at[pl.ds(sid * ROWS_PT + j * CH, CH)])
    pltpu.sync_copy(stage.at[pl.ds(0, ROWS_PT % CH)],
                    s_sh.at[pl.ds(sid * ROWS_PT + ROWS_PT - ROWS_PT % CH,
                                  ROWS_PT % CH)])

    @pl.when(sid == NS - 1)
    def _():
        pltpu.sync_copy(stage.at[pl.ds(0, NA2 - NS * ROWS_PT)],
                        s_sh.at[pl.ds(NS * ROWS_PT, NA2 - NS * ROWS_PT)])


def _copy_out_accum(s_sh, stage, out_hbm, cid, sid):
    def move(r0, n):
        pltpu.sync_copy(s_sh.at[pl.ds(r0, n)], stage.at[pl.ds(0, n)])
        pltpu.sync_copy(stage.at[pl.ds(0, n)], out_hbm.at[cid].at[pl.ds(r0, n)])

    for j in range(ROWS_PT // CH):
        move(sid * ROWS_PT + j * CH, CH)
    move(sid * ROWS_PT + ROWS_PT - ROWS_PT % CH, ROWS_PT % CH)

    @pl.when(sid == NS - 1)
    def _():
        move(NS * ROWS_PT, NA2 - NS * ROWS_PT)


@functools.cache
def _sc_scatter_kernel():
    bufs = []
    for _ in range(2):
        bufs += [
            pltpu.VMEM((CH,), jnp.int32),
            pltpu.VMEM((CH,), jnp.int32),
            pltpu.VMEM((CH, HID), jnp.float32),
        ]
    return pl.kernel(
        _sc_scatter_body,
        out_type=jax.ShapeDtypeStruct((NC, NA2, HID), jnp.float32),
        mesh=_sc_mesh(),
        scratch_types=bufs + [
            pltpu.VMEM_SHARED((NA2, HID), jnp.float32),
            pltpu.SemaphoreType.DMA,
            pltpu.SemaphoreType.DMA,
            pltpu.SemaphoreType.DMA,
            pltpu.SemaphoreType.DMA,
            pltpu.SemaphoreType.DMA,
            pltpu.SemaphoreType.DMA,
        ],
    )


def _sc_scatter_body(h_hbm, b2a_hbm, b2revb_hbm, out_hbm,
                     idxr0, dest0, rows0, idxr1, dest1, rows1, s_sh,
                     sd0, sr0, ss0, sd1, sr1, ss1):
    cid = lax.axis_index("c")
    sid = lax.axis_index("s")
    wid = sid * NC + cid
    s0 = dict(idxr=idxr0, dest=dest0, rows=rows0, sd=sd0, sr=sr0, ss=ss0)
    s1 = dict(idxr=idxr1, dest=dest1, rows=rows1, sd=sd1, sr=sr1, ss=ss1)

    def prefetch(k, s):
        base = wid * BPT + k * CH
        pltpu.sync_copy(b2revb_hbm.at[pl.ds(base, CH)], s['idxr'])
        pltpu.async_copy(b2a_hbm.at[s['idxr']], s['dest'], s['sd'])
        pltpu.async_copy(h_hbm.at[pl.ds(base, CH)], s['rows'], s['sr'])

    def wait_prefetch(s):
        pltpu.make_async_copy(b2a_hbm.at[s['idxr']], s['dest'], s['sd']).wait()
        pltpu.make_async_copy(h_hbm.at[pl.ds(0, CH)], s['rows'], s['sr']).wait()

    def issue_scatter(s):
        pltpu.async_copy(s['rows'], s_sh.at[s['dest']], s['ss'], add=True)

    def wait_scatter(s):
        pltpu.make_async_copy(s['rows'], s_sh.at[s['dest']], s['ss']).wait()

    _zero_accum(s_sh, rows0, sid)
    plsc.subcore_barrier()

    prefetch(0, s0)
    wait_prefetch(s0)
    issue_scatter(s0)
    prefetch(1, s1)

    def pair(i, c):
        wait_prefetch(s1)
        issue_scatter(s1)
        wait_scatter(s0)
        prefetch(2 * i + 2, s0)
        wait_prefetch(s0)
        issue_scatter(s0)
        wait_scatter(s1)
        prefetch(2 * i + 3, s1)
        return c
    lax.fori_loop(0, (NCHUNK - 2) // 2, pair, None)

    wait_prefetch(s1)
    issue_scatter(s1)
    wait_scatter(s0)
    wait_scatter(s1)

    plsc.subcore_barrier()
    _copy_out_accum(s_sh, rows0, out_hbm, cid, sid)


@functools.cache
def _sc_combine_kernel():
    bufs = []
    for _ in range(2):
        bufs += [
            pltpu.VMEM((CH,), jnp.int32),
            pltpu.VMEM((CH,), jnp.int32),
            pltpu.VMEM((CH,), jnp.int32),
            pltpu.VMEM((CH, HID), jnp.float32),
            pltpu.VMEM((CH, HID), jnp.float32),
            pltpu.VMEM((CH, HID), jnp.float32),
        ]
    return pl.kernel(
        _sc_combine_body,
        out_type=(
            jax.ShapeDtypeStruct((NBP, HID), jnp.float32),
            jax.ShapeDtypeStruct((NC, NA2, HID), jnp.float32),
        ),
        mesh=_sc_mesh(),
        scratch_types=bufs + [
            pltpu.VMEM_SHARED((NA2, HID), jnp.float32),
        ] + [pltpu.SemaphoreType.DMA] * 12,
    )


def _sc_combine_body(h_hbm, g_hbm, a_hbm, b2a_hbm, b2revb_hbm,
                     hnew_hbm, out_hbm,
                     idxa0, idxr0, dest0, a0, g0, h0,
                     idxa1, idxr1, dest1, a1, g1, h1,
                     s_sh,
                     sa0, sg0, sd0, sh0, sw0, ss0,
                     sa1, sg1, sd1, sh1, sw1, ss1):
    cid = lax.axis_index("c")
    sid = lax.axis_index("s")
    wid = sid * NC + cid
    s0 = dict(idxa=idxa0, idxr=idxr0, dest=dest0, a=a0, g=g0, h=h0,
              sa=sa0, sg=sg0, sd=sd0, sh=sh0, sw=sw0, ss=ss0)
    s1 = dict(idxa=idxa1, idxr=idxr1, dest=dest1, a=a1, g=g1, h=h1,
              sa=sa1, sg=sg1, sd=sd1, sh=sh1, sw=sw1, ss=ss1)

    def prefetch(k, s):
        base = wid * BPT + k * CH
        pltpu.sync_copy(b2a_hbm.at[pl.ds(base, CH)], s['idxa'])
        pltpu.sync_copy(b2revb_hbm.at[pl.ds(base, CH)], s['idxr'])
        pltpu.async_copy(a_hbm.at[s['idxa']], s['a'], s['sa'])
        pltpu.async_copy(g_hbm.at[s['idxr']], s['g'], s['sg'])
        pltpu.async_copy(b2a_hbm.at[s['idxr']], s['dest'], s['sd'])
        pltpu.async_copy(h_hbm.at[pl.ds(base, CH)], s['h'], s['sh'])

    def wait_prefetch(s):
        pltpu.make_async_copy(a_hbm.at[s['idxa']], s['a'], s['sa']).wait()
        pltpu.make_async_copy(g_hbm.at[s['idxr']], s['g'], s['sg']).wait()
        pltpu.make_async_copy(b2a_hbm.at[s['idxr']], s['dest'], s['sd']).wait()
        pltpu.make_async_copy(h_hbm.at[pl.ds(0, CH)], s['h'], s['sh']).wait()

    def compute(s):
        h, a, g = s['h'], s['a'], s['g']

        def row(r, c):
            for j in range(HID // 16):
                sl = pl.ds(j * 16, 16)
                h[r, sl] = jnp.maximum(h[r, sl] + a[r, sl] - g[r, sl], 0.0)
            return c
        lax.fori_loop(0, CH, row, None)

    def issue_writes(k, s):
        base = wid * BPT + k * CH
        pltpu.async_copy(s['h'], hnew_hbm.at[pl.ds(base, CH)], s['sw'])
        pltpu.async_copy(s['h'], s_sh.at[s['dest']], s['ss'], add=True)

    def wait_writes(s):
        pltpu.make_async_copy(s['h'], hnew_hbm.at[pl.ds(0, CH)], s['sw']).wait()
        pltpu.make_async_copy(s['h'], s_sh.at[s['dest']], s['ss']).wait()

    _zero_accum(s_sh, h0, sid)
    plsc.subcore_barrier()

    prefetch(0, s0)
    wait_prefetch(s0)
    compute(s0)
    issue_writes(0, s0)
    prefetch(1, s1)

    def pair(i, c):
        k1 = 2 * i + 1
        wait_prefetch(s1)
        compute(s1)
        issue_writes(k1, s1)
        wait_writes(s0)
        prefetch(k1 + 1, s0)

        wait_prefetch(s0)
        compute(s0)
        issue_writes(k1 + 1, s0)
        wait_writes(s1)
        prefetch(k1 + 2, s1)
        return c
    lax.fori_loop(0, (NCHUNK - 2) // 2, pair, None)

    wait_prefetch(s1)
    compute(s1)
    issue_writes(NCHUNK - 1, s1)
    wait_writes(s0)
    wait_writes(s1)

    plsc.subcore_barrier()
    _copy_out_accum(s_sh, h0, out_hbm, cid, sid)


def kernel(f_atoms, f_bonds, b2a, b2revb,
           Wi_w, Wi_b, Wh_w, Wh_b, Wo_w, Wo_b):
    wiT = Wi_w.T
    whT = Wh_w.T
    wo1T = Wo_w[:, :AFD].T
    wo2T = Wo_w[:, AFD:].T
    bi = Wi_b.reshape(1, HID)
    bh = Wh_b.reshape(1, HID)
    bo = Wo_b.reshape(1, HID)

    b2a_p = jnp.concatenate([b2a, jnp.full((NPAD,), DUMMY, jnp.int32)])
    b2revb_p = jnp.concatenate([b2revb, jnp.full((NPAD,), NB, jnp.int32)])

    h_bonds = _tc_init(f_bonds, wiT, bi)
    s_part = _sc_scatter_kernel()(h_bonds, b2a_p, b2revb_p)
    for _ in range(2):
        a = _tc_a(s_part, whT, bh)
        g = _tc_g(h_bonds, whT)
        h_bonds, s_part = _sc_combine_kernel()(h_bonds, g, a, b2a_p, b2revb_p)
    h_atoms = _tc_final(f_atoms, s_part, wo1T, wo2T, bo)
    return (h_atoms, h_bonds[:NB])


# speedup vs baseline: 3.2096x; 2.0964x over previous
import functools

import jax
import jax.numpy as jnp
from jax import lax
from jax.experimental import pallas as pl
from jax.experimental.pallas import tpu as pltpu
from jax.experimental.pallas import tpu_sc as plsc

NB = 320000
NA = 10000
HID = 128
BFD = 16
AFD = 128

NC = 2
NS = 16
NW = NC * NS

CH = 64
NCH_LO = 156
NCH_HI = 158

ROWS_PT = 624


@functools.cache
def _sc_mesh():
    return plsc.VectorSubcoreMesh(
        core_axis_name="c", subcore_axis_name="s",
        num_cores=NC, num_subcores=NS)


def _init_body(fb_ref, w_ref, b_ref, out_ref):
    acc = jnp.dot(fb_ref[...], w_ref[...], preferred_element_type=jnp.float32)
    out_ref[...] = jnp.maximum(acc + b_ref[...], 0.0)


def _tc_init(f_bonds, wiT, bi):
    bt = 4000
    return pl.pallas_call(
        _init_body,
        grid=(NB // bt,),
        in_specs=[
            pl.BlockSpec((bt, BFD), lambda i: (i, 0)),
            pl.BlockSpec((BFD, HID), lambda i: (0, 0)),
            pl.BlockSpec((1, HID), lambda i: (0, 0)),
        ],
        out_specs=pl.BlockSpec((bt, HID), lambda i: (i, 0)),
        out_shape=jax.ShapeDtypeStruct((NB, HID), jnp.float32),
    )(f_bonds, wiT, bi)


def _g_body(h_ref, w_ref, out_ref):
    out_ref[...] = jnp.dot(h_ref[...], w_ref[...],
                           preferred_element_type=jnp.float32)


def _tc_g(h_bonds, whT):
    bt = 4000
    return pl.pallas_call(
        _g_body,
        grid=(NB // bt,),
        in_specs=[
            pl.BlockSpec((bt, HID), lambda i: (i, 0)),
            pl.BlockSpec((HID, HID), lambda i: (0, 0)),
        ],
        out_specs=pl.BlockSpec((bt, HID), lambda i: (i, 0)),
        out_shape=jax.ShapeDtypeStruct((NB, HID), jnp.float32),
    )(h_bonds, whT)


def _a_body(sp_ref, w_ref, b_ref, out_ref):
    s = sp_ref[0] + sp_ref[1]
    out_ref[...] = jnp.dot(s, w_ref[...],
                           preferred_element_type=jnp.float32) + b_ref[...]


def _tc_a(s_part, whT, bh):
    return pl.pallas_call(
        _a_body,
        out_shape=jax.ShapeDtypeStruct((NA, HID), jnp.float32),
    )(s_part, whT, bh)


def _final_body(fa_ref, sp_ref, w1_ref, w2_ref, b_ref, out_ref):
    s = sp_ref[0] + sp_ref[1]
    acc = jnp.dot(fa_ref[...], w1_ref[...], preferred_element_type=jnp.float32)
    acc += jnp.dot(s, w2_ref[...], preferred_element_type=jnp.float32)
    out_ref[...] = jnp.maximum(acc + b_ref[...], 0.0)


def _tc_final(f_atoms, s_part, wo1T, wo2T, bo):
    return pl.pallas_call(
        _final_body,
        out_shape=jax.ShapeDtypeStruct((NA, HID), jnp.float32),
    )(f_atoms, s_part, wo1T, wo2T, bo)


def _tile_chunks(wid):
    cstart = NCH_LO * wid + 2 * jnp.minimum(wid, 4)
    nchunks = jnp.where(wid < 4, NCH_HI, NCH_LO)
    return cstart, nchunks


def _zero_accum(s_sh, stage, sid):
    def fill_row(r, c):
        for j in range(HID // 16):
            stage[r, pl.ds(j * 16, 16)] = jnp.zeros((16,), jnp.float32)
        return c
    lax.fori_loop(0, CH, fill_row, None)
    for j in range(ROWS_PT // CH):
        pltpu.sync_copy(stage, s_sh.at[pl.ds(sid * ROWS_PT + j * CH, CH)])
    rem = ROWS_PT % CH
    pltpu.sync_copy(stage.at[pl.ds(0, rem)],
                    s_sh.at[pl.ds(sid * ROWS_PT + ROWS_PT - rem, rem)])

    @pl.when(sid == NS - 1)
    def _():
        tail = NA - NS * ROWS_PT
        pltpu.sync_copy(stage.at[pl.ds(0, tail)],
                        s_sh.at[pl.ds(NS * ROWS_PT, tail)])


def _copy_out_accum(s_sh, stage, out_hbm, cid, sid):
    def move(r0, n):
        pltpu.sync_copy(s_sh.at[pl.ds(r0, n)], stage.at[pl.ds(0, n)])
        pltpu.sync_copy(stage.at[pl.ds(0, n)], out_hbm.at[cid].at[pl.ds(r0, n)])

    for j in range(ROWS_PT // CH):
        move(sid * ROWS_PT + j * CH, CH)
    rem = ROWS_PT % CH
    move(sid * ROWS_PT + ROWS_PT - rem, rem)

    @pl.when(sid == NS - 1)
    def _():
        move(NS * ROWS_PT, NA - NS * ROWS_PT)


@functools.cache
def _sc_scatter_kernel():
    bufs = []
    for _ in range(2):
        bufs += [
            pltpu.VMEM((CH,), jnp.int32),
            pltpu.VMEM((CH,), jnp.int32),
            pltpu.VMEM((CH, HID), jnp.float32),
        ]
    return pl.kernel(
        _sc_scatter_body,
        out_type=jax.ShapeDtypeStruct((NC, NA, HID), jnp.float32),
        mesh=_sc_mesh(),
        scratch_types=bufs + [
            pltpu.VMEM_SHARED((NA, HID), jnp.float32),
        ] + [pltpu.SemaphoreType.DMA] * 6,
    )


def _sc_scatter_body(h_hbm, b2a_hbm, b2revb_hbm, out_hbm,
                     idxr0, dest0, rows0, idxr1, dest1, rows1, s_sh,
                     sd0, sr0, ss0, sd1, sr1, ss1):
    cid = lax.axis_index("c")
    sid = lax.axis_index("s")
    wid = sid * NC + cid
    cstart, nchunks = _tile_chunks(wid)
    s0 = dict(idxr=idxr0, dest=dest0, rows=rows0, sd=sd0, sr=sr0, ss=ss0)
    s1 = dict(idxr=idxr1, dest=dest1, rows=rows1, sd=sd1, sr=sr1, ss=ss1)

    def prefetch(k, s):
        base = (cstart + k) * CH
        pltpu.sync_copy(b2revb_hbm.at[pl.ds(base, CH)], s['idxr'])
        pltpu.async_copy(b2a_hbm.at[s['idxr']], s['dest'], s['sd'])
        pltpu.async_copy(h_hbm.at[pl.ds(base, CH)], s['rows'], s['sr'])

    def wait_prefetch(s):
        pltpu.make_async_copy(b2a_hbm.at[s['idxr']], s['dest'], s['sd']).wait()
        pltpu.make_async_copy(h_hbm.at[pl.ds(0, CH)], s['rows'], s['sr']).wait()

    def issue_scatter(s):
        pltpu.async_copy(s['rows'], s_sh.at[s['dest']], s['ss'], add=True)

    def wait_scatter(s):
        pltpu.make_async_copy(s['rows'], s_sh.at[s['dest']], s['ss']).wait()

    _zero_accum(s_sh, rows0, sid)
    plsc.subcore_barrier()

    prefetch(0, s0)
    prefetch(1, s1)
    wait_prefetch(s0)
    issue_scatter(s0)

    def pair(i, c):
        k1 = 2 * i + 1
        wait_scatter(s0)
        prefetch(k1 + 1, s0)
        wait_prefetch(s1)
        issue_scatter(s1)

        wait_scatter(s1)
        prefetch(k1 + 2, s1)
        wait_prefetch(s0)
        issue_scatter(s0)
        return c
    lax.fori_loop(0, (nchunks - 2) // 2, pair, None)

    wait_prefetch(s1)
    issue_scatter(s1)
    wait_scatter(s0)
    wait_scatter(s1)

    plsc.subcore_barrier()
    _copy_out_accum(s_sh, rows0, out_hbm, cid, sid)


@functools.cache
def _sc_combine_kernel():
    bufs = []
    for _ in range(2):
        bufs += [
            pltpu.VMEM((CH,), jnp.int32),
            pltpu.VMEM((CH,), jnp.int32),
            pltpu.VMEM((CH,), jnp.int32),
            pltpu.VMEM((CH, HID), jnp.float32),
            pltpu.VMEM((CH, HID), jnp.float32),
            pltpu.VMEM((CH, HID), jnp.float32),
        ]
    return pl.kernel(
        _sc_combine_body,
        out_type=(
            jax.ShapeDtypeStruct((NB, HID), jnp.float32),
            jax.ShapeDtypeStruct((NC, NA, HID), jnp.float32),
        ),
        mesh=_sc_mesh(),
        scratch_types=bufs + [
            pltpu.VMEM_SHARED((NA, HID), jnp.float32),
        ] + [pltpu.SemaphoreType.DMA] * 12,
    )


def _sc_combine_body(h_hbm, g_hbm, a_hbm, b2a_hbm, b2revb_hbm,
                     hnew_hbm, out_hbm,
                     idxa0, idxr0, dest0, a0, g0, h0,
                     idxa1, idxr1, dest1, a1, g1, h1,
                     s_sh,
                     sa0, sg0, sd0, sh0, sw0, ss0,
                     sa1, sg1, sd1, sh1, sw1, ss1):
    cid = lax.axis_index("c")
    sid = lax.axis_index("s")
    wid = sid * NC + cid
    cstart, nchunks = _tile_chunks(wid)
    s0 = dict(idxa=idxa0, idxr=idxr0, dest=dest0, a=a0, g=g0, h=h0,
              sa=sa0, sg=sg0, sd=sd0, sh=sh0, sw=sw0, ss=ss0)
    s1 = dict(idxa=idxa1, idxr=idxr1, dest=dest1, a=a1, g=g1, h=h1,
              sa=sa1, sg=sg1, sd=sd1, sh=sh1, sw=sw1, ss=ss1)

    def prefetch(k, s):
        base = (cstart + k) * CH
        pltpu.sync_copy(b2a_hbm.at[pl.ds(base, CH)], s['idxa'])
        pltpu.sync_copy(b2revb_hbm.at[pl.ds(base, CH)], s['idxr'])
        pltpu.async_copy(a_hbm.at[s['idxa']], s['a'], s['sa'])
        pltpu.async_copy(g_hbm.at[s['idxr']], s['g'], s['sg'])
        pltpu.async_copy(b2a_hbm.at[s['idxr']], s['dest'], s['sd'])
        pltpu.async_copy(h_hbm.at[pl.ds(base, CH)], s['h'], s['sh'])

    def wait_prefetch(s):
        pltpu.make_async_copy(a_hbm.at[s['idxa']], s['a'], s['sa']).wait()
        pltpu.make_async_copy(g_hbm.at[s['idxr']], s['g'], s['sg']).wait()
        pltpu.make_async_copy(b2a_hbm.at[s['idxr']], s['dest'], s['sd']).wait()
        pltpu.make_async_copy(h_hbm.at[pl.ds(0, CH)], s['h'], s['sh']).wait()

    def compute(s):
        h, a, g = s['h'], s['a'], s['g']

        def row(r, c):
            for j in range(HID // 16):
                sl = pl.ds(j * 16, 16)
                h[r, sl] = jnp.maximum(h[r, sl] + a[r, sl] - g[r, sl], 0.0)
            return c
        lax.fori_loop(0, CH, row, None)

    def issue_writes(k, s):
        base = (cstart + k) * CH
        pltpu.async_copy(s['h'], hnew_hbm.at[pl.ds(base, CH)], s['sw'])
        pltpu.async_copy(s['h'], s_sh.at[s['dest']], s['ss'], add=True)

    def wait_writes(s):
        pltpu.make_async_copy(s['h'], hnew_hbm.at[pl.ds(0, CH)], s['sw']).wait()
        pltpu.make_async_copy(s['h'], s_sh.at[s['dest']], s['ss']).wait()

    _zero_accum(s_sh, h0, sid)
    plsc.subcore_barrier()

    prefetch(0, s0)
    prefetch(1, s1)
    wait_prefetch(s0)
    compute(s0)
    issue_writes(0, s0)

    def pair(i, c):
        k1 = 2 * i + 1
        wait_writes(s0)
        prefetch(k1 + 1, s0)
        wait_prefetch(s1)
        compute(s1)
        issue_writes(k1, s1)

        wait_writes(s1)
        prefetch(k1 + 2, s1)
        wait_prefetch(s0)
        compute(s0)
        issue_writes(k1 + 1, s0)
        return c
    lax.fori_loop(0, (nchunks - 2) // 2, pair, None)

    wait_prefetch(s1)
    compute(s1)
    issue_writes(nchunks - 1, s1)
    wait_writes(s0)
    wait_writes(s1)

    plsc.subcore_barrier()
    _copy_out_accum(s_sh, h0, out_hbm, cid, sid)


def kernel(f_atoms, f_bonds, b2a, b2revb,
           Wi_w, Wi_b, Wh_w, Wh_b, Wo_w, Wo_b):
    wiT = Wi_w.T
    whT = Wh_w.T
    wo1T = Wo_w[:, :AFD].T
    wo2T = Wo_w[:, AFD:].T
    bi = Wi_b.reshape(1, HID)
    bh = Wh_b.reshape(1, HID)
    bo = Wo_b.reshape(1, HID)

    h_bonds = _tc_init(f_bonds, wiT, bi)
    s_part = _sc_scatter_kernel()(h_bonds, b2a, b2revb)
    for _ in range(2):
        a = _tc_a(s_part, whT, bh)
        g = _tc_g(h_bonds, whT)
        h_bonds, s_part = _sc_combine_kernel()(h_bonds, g, a, b2a, b2revb)
    h_atoms = _tc_final(f_atoms, s_part, wo1T, wo2T, bo)
    return (h_atoms, h_bonds)
